# baseline (speedup 1.0000x reference)
import functools
import math

import jax
import jax.numpy as jnp
from jax import lax
from jax.experimental import pallas as pl
from jax.experimental.pallas import tpu as pltpu
from jax.experimental.pallas import tpu_sc as plsc

N = 10000
E = 320000
H = 128
F = 128
G = 50
CUTOFF = 8.0
NUM_INT = 3
NUM_GRAPHS = 64
NZ = 100

NC = 2
NS = 16
NW = NC * NS
EPT = E // NW
CH = 128
NCH = E // CH
CPT = NCH // NW
RPT = 632
NPAD = RPT * NS

HP = H // 2

_mesh = plsc.VectorSubcoreMesh(core_axis_name="c", subcore_axis_name="s")
_sc_params = pltpu.CompilerParams(needs_layout_passes=False)


def _ssp(x):
    return jnp.maximum(x, 0.0) + jnp.log(1.0 + jnp.exp(-jnp.abs(x))) - math.log(2.0)


@functools.partial(
    pl.kernel,
    out_type=jax.ShapeDtypeStruct((E,), jnp.float32),
    mesh=_mesh,
    scratch_types=[
        pltpu.VMEM((3 * N,), jnp.float32),
        pltpu.VMEM((EPT,), jnp.int32),
        pltpu.VMEM((EPT,), jnp.int32),
        pltpu.VMEM((3 * EPT,), jnp.float32),
        pltpu.VMEM((EPT,), jnp.float32),
    ],
    compiler_params=_sc_params,
)
def _geom(pos_hbm, row_hbm, col_hbm, shift_hbm, ew2_hbm,
          pos_v, row_v, col_v, shift_v, ew2_v):
    wid = lax.axis_index("s") * NC + lax.axis_index("c")
    base = wid * EPT
    pltpu.sync_copy(pos_hbm, pos_v)
    pltpu.sync_copy(row_hbm.at[pl.ds(base, EPT)], row_v)
    pltpu.sync_copy(col_hbm.at[pl.ds(base, EPT)], col_v)
    pltpu.sync_copy(shift_hbm.at[pl.ds(3 * base, 3 * EPT)], shift_v)
    lanes = lax.iota(jnp.int32, 16)

    def body(k, carry):
        rv = row_v[pl.ds(k * 16, 16)] * 3
        cv = col_v[pl.ds(k * 16, 16)] * 3
        sbase = k * 48 + lanes * 3
        acc = None
        for j in range(3):
            pr = plsc.load_gather(pos_v, [rv + j])
            pc = plsc.load_gather(pos_v, [cv + j])
            sj = plsc.load_gather(shift_v, [sbase + j])
            d = pr - (pc + sj)
            acc = d * d if acc is None else acc + d * d
        ew2_v[pl.ds(k * 16, 16)] = acc
        return carry

    lax.fori_loop(0, EPT // 16, body, 0)
    pltpu.sync_copy(ew2_v, ew2_hbm.at[pl.ds(base, EPT)])


@functools.partial(
    pl.kernel,
    out_type=jax.ShapeDtypeStruct((NC, NPAD, H), jnp.float32),
    mesh=_mesh,
    scratch_types=[
        pltpu.VMEM((2, CH), jnp.int32),
        pltpu.VMEM((2, CH), jnp.int32),
        pltpu.VMEM((CH, H), jnp.float32),
        pltpu.VMEM((CH, H), jnp.float32),
        pltpu.VMEM((CH // 2, H), jnp.uint32),
        pltpu.VMEM((CH // 2, H), jnp.uint32),
        pltpu.VMEM_SHARED((NPAD, H), jnp.float32),
        pltpu.SemaphoreType.DMA,
        pltpu.SemaphoreType.DMA,
        pltpu.SemaphoreType.DMA,
        pltpu.SemaphoreType.DMA,
        pltpu.SemaphoreType.DMA,
        pltpu.SemaphoreType.DMA,
        pltpu.SemaphoreType.DMA,
        pltpu.SemaphoreType.DMA,
    ],
    compiler_params=_sc_params,
)
def _conv(xl_hbm, w_hbm, row_hbm, col_hbm, out_hbm,
          row_v, col_v, xga, xgb, wva, wvb, acc,
          rs0, rs1, cs0, cs1, gs0, gs1, ws0, ws1):
    cid = lax.axis_index("c")
    sid = lax.axis_index("s")
    wid = sid * NC + cid
    xg = (xga, xgb)
    wv = (wva, wvb)
    rs = (rs0, rs1)
    cs = (cs0, cs1)
    gs = (gs0, gs1)
    ws = (ws0, ws1)

    def zbody(r, carry):
        for f in range(H // 16):
            xga[r, pl.ds(f * 16, 16)] = jnp.zeros((16,), jnp.float32)
        return carry

    lax.fori_loop(0, CH, zbody, 0)
    for t in range(RPT // CH):
        pltpu.sync_copy(xga, acc.at[pl.ds(sid * RPT + t * CH, CH)])
    pltpu.sync_copy(xga.at[pl.ds(0, RPT - (RPT // CH) * CH)],
                    acc.at[pl.ds(sid * RPT + (RPT // CH) * CH,
                                 RPT - (RPT // CH) * CH)])
    plsc.subcore_barrier()

    cstart = wid * CPT + 2 * jnp.minimum(wid, 2)
    cnum = CPT + jnp.where(wid < 2, 2, 0)
    cmax = cstart + cnum - 1

    def issue_idx(c, b):
        pltpu.async_copy(row_hbm.at[pl.ds(c * CH, CH)], row_v.at[b], rs[b])
        pltpu.async_copy(col_hbm.at[pl.ds(c * CH, CH)], col_v.at[b], cs[b])

    def wait_idx(b):
        pltpu.make_async_copy(row_hbm.at[pl.ds(0, CH)], row_v.at[b], rs[b]).wait()
        pltpu.make_async_copy(col_hbm.at[pl.ds(0, CH)], col_v.at[b], cs[b]).wait()

    def issue_fetch(c, b):
        pltpu.async_copy(xl_hbm.at[row_v.at[b]], xg[b], gs[b])
        pltpu.async_copy(w_hbm.at[pl.ds(c * (CH // 2), CH // 2)], wv[b], ws[b])

    def wait_fetch(b):
        pltpu.make_async_copy(xl_hbm.at[row_v.at[b]], xg[b], gs[b]).wait()
        pltpu.make_async_copy(w_hbm.at[pl.ds(0, CH // 2)], wv[b], ws[b]).wait()

    issue_idx(cstart, 0)
    wait_idx(0)
    issue_fetch(cstart, 0)
    issue_idx(cstart + 1, 1)

    def phase(c, b):
        nb = 1 - b
        wait_idx(nb)
        issue_fetch(jnp.minimum(c + 1, cmax), nb)
        wait_fetch(b)

        hi_mask = jnp.full((16,), 0xFFFF0000, jnp.uint32)

        def mul(r2, c2):
            for f in range(H // 16):
                wu = wv[b][r2, pl.ds(f * 16, 16)]
                we = plsc.bitcast(wu << 16, jnp.float32)
                wo = plsc.bitcast(wu & hi_mask, jnp.float32)
                xg[b][r2, pl.ds(f * 16, 16)] = (
                    xg[b][r2, pl.ds(f * 16, 16)] * we)
                xg[b][r2 + CH // 2, pl.ds(f * 16, 16)] = (
                    xg[b][r2 + CH // 2, pl.ds(f * 16, 16)] * wo)
            return c2

        lax.fori_loop(0, CH // 2, mul, 0)
        pltpu.sync_copy(xg[b], acc.at[col_v.at[b]], add=True)
        issue_idx(jnp.minimum(c + 2, cmax), b)

    def pair(t, carry):
        c = cstart + t * 2
        phase(c, 0)
        phase(c + 1, 1)
        return carry

    lax.fori_loop(0, cnum // 2, pair, 0)
    wait_fetch(0)
    wait_idx(1)
    plsc.subcore_barrier()
    pltpu.sync_copy(acc.at[pl.ds(sid * RPT, RPT)],
                    out_hbm.at[cid, pl.ds(sid * RPT, RPT)])


BE = 1280
NBE = E // BE


def _w_body(ew2_ref, w1_ref, b1_ref, w2_ref, b2_ref, o_ref):
    ew2 = ew2_ref[...]
    ew = jnp.sqrt(ew2)
    step = CUTOFF / (G - 1)
    offs = lax.broadcasted_iota(jnp.int32, (1, G), 1).astype(jnp.float32) * step
    coeff = -0.5 / (step * step)
    ea = jnp.exp(coeff * (ew - offs) ** 2)
    cc = 0.5 * (jnp.cos(ew * (math.pi / CUTOFF)) + 1.0)
    t = _ssp(ea @ w1_ref[...] + b1_ref[...])
    t = (t @ w2_ref[...] + b2_ref[...]) * cc
    tb = lax.bitcast_convert_type(t.astype(jnp.bfloat16), jnp.uint16)
    for cl in range(BE // CH):
        lo = tb[cl * CH:cl * CH + CH // 2, :].astype(jnp.uint32)
        hi = tb[cl * CH + CH // 2:(cl + 1) * CH, :].astype(jnp.uint32)
        o_ref[pl.ds(cl * (CH // 2), CH // 2), :] = lo | (hi << 16)


def _wkern(ew2, w1, b1, w2, b2):
    return pl.pallas_call(
        _w_body,
        grid=(NBE,),
        in_specs=[
            pl.BlockSpec((BE, 1), lambda j: (j, 0)),
            pl.BlockSpec((G, F), lambda j: (0, 0)),
            pl.BlockSpec((1, F), lambda j: (0, 0)),
            pl.BlockSpec((F, F), lambda j: (0, 0)),
            pl.BlockSpec((1, F), lambda j: (0, 0)),
        ],
        out_specs=pl.BlockSpec((BE // 2, H), lambda j: (j, 0)),
        out_shape=jax.ShapeDtypeStruct((E // 2, H), jnp.uint32),
    )(ew2, w1, b1, w2, b2)


BN = 1000
NBN = N // BN


def _prep_body(z_ref, emb_ref, lin1_ref, h_ref, xl_ref):
    z = z_ref[...]
    oh = (z == lax.broadcasted_iota(jnp.int32, (1, NZ), 1)).astype(jnp.float32)
    h = oh @ emb_ref[...]
    h_ref[...] = h
    xl_ref[...] = h @ lin1_ref[...]


def _prep(z, emb, lin1):
    return pl.pallas_call(
        _prep_body,
        grid=(NBN,),
        in_specs=[
            pl.BlockSpec((BN, 1), lambda j: (j, 0)),
            pl.BlockSpec((NZ, H), lambda j: (0, 0)),
            pl.BlockSpec((H, F), lambda j: (0, 0)),
        ],
        out_specs=[pl.BlockSpec((BN, H), lambda j: (j, 0))] * 2,
        out_shape=[jax.ShapeDtypeStruct((N, H), jnp.float32)] * 2,
    )(z, emb, lin1)


def _update_body(last, p_ref, h_ref, w2_ref, b2_ref, bw_ref, bb_ref,
                 lin1_ref, h_out_ref, xl_out_ref=None):
    agg = p_ref[0] + p_ref[1]
    x = agg @ w2_ref[...] + b2_ref[...]
    x = _ssp(x)
    x = x @ bw_ref[...] + bb_ref[...]
    hn = h_ref[...] + x
    h_out_ref[...] = hn
    if not last:
        xl_out_ref[...] = hn @ lin1_ref[...]


def _update(p, h, w2, b2, bw, bb, lin1, last):
    out_shape = [jax.ShapeDtypeStruct((N, H), jnp.float32)]
    out_specs = [pl.BlockSpec((BN, H), lambda j: (j, 0))]
    if not last:
        out_shape.append(jax.ShapeDtypeStruct((N, H), jnp.float32))
        out_specs.append(pl.BlockSpec((BN, H), lambda j: (j, 0)))
    return pl.pallas_call(
        functools.partial(_update_body, last),
        grid=(NBN,),
        in_specs=[
            pl.BlockSpec((NC, BN, H), lambda j: (0, j, 0)),
            pl.BlockSpec((BN, H), lambda j: (j, 0)),
            pl.BlockSpec((F, H), lambda j: (0, 0)),
            pl.BlockSpec((1, H), lambda j: (0, 0)),
            pl.BlockSpec((H, H), lambda j: (0, 0)),
            pl.BlockSpec((1, H), lambda j: (0, 0)),
            pl.BlockSpec((H, F), lambda j: (0, 0)),
        ],
        out_specs=out_specs,
        out_shape=out_shape,
    )(p, h, w2, b2, bw, bb, lin1)


def _head_body(h_ref, b_ref, o1w_ref, o1b_ref, o2w_ref, o2b_ref,
               out_ref, s_acc, c_acc):
    j = pl.program_id(0)

    @pl.when(j == 0)
    def _():
        s_acc[...] = jnp.zeros_like(s_acc)
        c_acc[...] = jnp.zeros_like(c_acc)

    t = _ssp(h_ref[...] @ o1w_ref[...] + o1b_ref[...])
    bt = b_ref[0]
    oh = (lax.broadcasted_iota(jnp.int32, (NUM_GRAPHS, 1), 0) == bt
          ).astype(jnp.float32)
    s_acc[...] += oh @ t
    c_acc[...] += jnp.sum(oh, axis=1, keepdims=True)

    @pl.when(j == pl.num_programs(0) - 1)
    def _():
        m = s_acc[...] / jnp.maximum(c_acc[...], 1.0)
        out_ref[...] = m @ o2w_ref[...] + o2b_ref[...]


def _head(h, batch3, o1w, o1b, o2w, o2b):
    return pl.pallas_call(
        _head_body,
        grid=(NBN,),
        in_specs=[
            pl.BlockSpec((BN, H), lambda j: (j, 0)),
            pl.BlockSpec((1, 1, BN), lambda j: (j, 0, 0)),
            pl.BlockSpec((H, H // 2), lambda j: (0, 0)),
            pl.BlockSpec((1, H // 2), lambda j: (0, 0)),
            pl.BlockSpec((H // 2, 1), lambda j: (0, 0)),
            pl.BlockSpec((1, 1), lambda j: (0, 0)),
        ],
        out_specs=pl.BlockSpec((NUM_GRAPHS, 1), lambda j: (0, 0)),
        out_shape=jax.ShapeDtypeStruct((NUM_GRAPHS, 1), jnp.float32),
        scratch_shapes=[
            pltpu.VMEM((NUM_GRAPHS, H // 2), jnp.float32),
            pltpu.VMEM((NUM_GRAPHS, 1), jnp.float32),
        ],
    )(h, batch3, o1w, o1b, o2w, o2b)


def kernel(z, pos, edge_index, edge_shift, batch, emb, mlp_w1, mlp_b1,
           mlp_w2, mlp_b2, conv_lin1_w, conv_lin2_w, conv_lin2_b,
           blk_lin_w, blk_lin_b, out1_w, out1_b, out2_w, out2_b):
    row = edge_index[0]
    col = edge_index[1]
    ew2 = _geom(pos.reshape(-1), row, col, edge_shift.reshape(-1))
    ew2 = ew2.reshape(E, 1)
    h, xl = _prep(z.reshape(N, 1).astype(jnp.int32), emb, conv_lin1_w[0])
    for i in range(NUM_INT):
        w_i = _wkern(ew2, mlp_w1[i], mlp_b1[i].reshape(1, F),
                     mlp_w2[i], mlp_b2[i].reshape(1, F))
        p = _conv(xl, w_i, row, col)
        last = i == NUM_INT - 1
        res = _update(p, h, conv_lin2_w[i], conv_lin2_b[i].reshape(1, H),
                      blk_lin_w[i], blk_lin_b[i].reshape(1, H),
                      conv_lin1_w[(i + 1) % NUM_INT], last)
        if last:
            h = res[0]
        else:
            h, xl = res
    return _head(h, batch.reshape(NBN, 1, BN).astype(jnp.int32), out1_w,
                 out1_b.reshape(1, H // 2), out2_w, out2_b.reshape(1, 1))


# speedup vs baseline: 1.9773x; 1.9773x over previous
import functools
import math

import jax
import jax.numpy as jnp
from jax import lax
from jax.experimental import pallas as pl
from jax.experimental.pallas import tpu as pltpu
from jax.experimental.pallas import tpu_sc as plsc

N = 10000
E = 320000
H = 128
F = 128
G = 50
CUTOFF = 8.0
NUM_INT = 3
NUM_GRAPHS = 64
NZ = 100

NC = 2
NS = 16
NW = NC * NS
EPT = E // NW
CH = 128
NCH = E // CH
CPT = NCH // NW
RPT = 632
NPAD = RPT * NS

HP = H // 2

_mesh = plsc.VectorSubcoreMesh(core_axis_name="c", subcore_axis_name="s")
_sc_params = pltpu.CompilerParams(needs_layout_passes=False)


def _ssp(x):
    return jnp.maximum(x, 0.0) + jnp.log(1.0 + jnp.exp(-jnp.abs(x))) - math.log(2.0)


@functools.partial(
    pl.kernel,
    out_type=jax.ShapeDtypeStruct((E,), jnp.float32),
    mesh=_mesh,
    scratch_types=[
        pltpu.VMEM((3 * N,), jnp.float32),
        pltpu.VMEM((EPT,), jnp.int32),
        pltpu.VMEM((EPT,), jnp.int32),
        pltpu.VMEM((3 * EPT,), jnp.float32),
        pltpu.VMEM((EPT,), jnp.float32),
    ],
    compiler_params=_sc_params,
)
def _geom(pos_hbm, row_hbm, col_hbm, shift_hbm, ew2_hbm,
          pos_v, row_v, col_v, shift_v, ew2_v):
    wid = lax.axis_index("s") * NC + lax.axis_index("c")
    base = wid * EPT
    pltpu.sync_copy(pos_hbm, pos_v)
    pltpu.sync_copy(row_hbm.at[pl.ds(base, EPT)], row_v)
    pltpu.sync_copy(col_hbm.at[pl.ds(base, EPT)], col_v)
    pltpu.sync_copy(shift_hbm.at[pl.ds(3 * base, 3 * EPT)], shift_v)
    lanes = lax.iota(jnp.int32, 16)

    def body(k, carry):
        rv = row_v[pl.ds(k * 16, 16)] * 3
        cv = col_v[pl.ds(k * 16, 16)] * 3
        sbase = k * 48 + lanes * 3
        acc = None
        for j in range(3):
            pr = plsc.load_gather(pos_v, [rv + j])
            pc = plsc.load_gather(pos_v, [cv + j])
            sj = plsc.load_gather(shift_v, [sbase + j])
            d = pr - (pc + sj)
            acc = d * d if acc is None else acc + d * d
        ew2_v[pl.ds(k * 16, 16)] = acc
        return carry

    lax.fori_loop(0, EPT // 16, body, 0)
    pltpu.sync_copy(ew2_v, ew2_hbm.at[pl.ds(base, EPT)])


@functools.partial(
    pl.kernel,
    out_type=jax.ShapeDtypeStruct((NC, NPAD, H), jnp.float32),
    mesh=_mesh,
    scratch_types=[
        pltpu.VMEM((2, CH), jnp.int32),
        pltpu.VMEM((2, CH), jnp.int32),
        pltpu.VMEM((CH, H), jnp.float32),
        pltpu.VMEM((CH, H), jnp.float32),
        pltpu.VMEM((CH // 2, H), jnp.uint32),
        pltpu.VMEM((CH // 2, H), jnp.uint32),
        pltpu.VMEM_SHARED((NPAD, H), jnp.float32),
        pltpu.SemaphoreType.DMA,
        pltpu.SemaphoreType.DMA,
        pltpu.SemaphoreType.DMA,
        pltpu.SemaphoreType.DMA,
        pltpu.SemaphoreType.DMA,
        pltpu.SemaphoreType.DMA,
        pltpu.SemaphoreType.DMA,
        pltpu.SemaphoreType.DMA,
    ],
    compiler_params=_sc_params,
)
def _conv(xl_hbm, w_hbm, row_hbm, col_hbm, out_hbm,
          row_v, col_v, xga, xgb, wva, wvb, acc,
          rs0, rs1, cs0, cs1, gs0, gs1, ws0, ws1):
    cid = lax.axis_index("c")
    sid = lax.axis_index("s")
    wid = sid * NC + cid
    xg = (xga, xgb)
    wv = (wva, wvb)
    rs = (rs0, rs1)
    cs = (cs0, cs1)
    gs = (gs0, gs1)
    ws = (ws0, ws1)

    def zbody(r, carry):
        for f in range(H // 16):
            xga[r, pl.ds(f * 16, 16)] = jnp.zeros((16,), jnp.float32)
        return carry

    lax.fori_loop(0, CH, zbody, 0)
    for t in range(RPT // CH):
        pltpu.sync_copy(xga, acc.at[pl.ds(sid * RPT + t * CH, CH)])
    pltpu.sync_copy(xga.at[pl.ds(0, RPT - (RPT // CH) * CH)],
                    acc.at[pl.ds(sid * RPT + (RPT // CH) * CH,
                                 RPT - (RPT // CH) * CH)])
    plsc.subcore_barrier()

    cstart = wid * CPT + 2 * jnp.minimum(wid, 2)
    cnum = CPT + jnp.where(wid < 2, 2, 0)
    cmax = cstart + cnum - 1

    def issue_idx(c, b):
        pltpu.async_copy(row_hbm.at[pl.ds(c * CH, CH)], row_v.at[b], rs[b])
        pltpu.async_copy(col_hbm.at[pl.ds(c * CH, CH)], col_v.at[b], cs[b])

    def wait_idx(b):
        pltpu.make_async_copy(row_hbm.at[pl.ds(0, CH)], row_v.at[b], rs[b]).wait()
        pltpu.make_async_copy(col_hbm.at[pl.ds(0, CH)], col_v.at[b], cs[b]).wait()

    def issue_fetch(c, b):
        pltpu.async_copy(xl_hbm.at[row_v.at[b]], xg[b], gs[b])
        pltpu.async_copy(w_hbm.at[pl.ds(c * (CH // 2), CH // 2)], wv[b], ws[b])

    def wait_fetch(b):
        pltpu.make_async_copy(xl_hbm.at[row_v.at[b]], xg[b], gs[b]).wait()
        pltpu.make_async_copy(w_hbm.at[pl.ds(0, CH // 2)], wv[b], ws[b]).wait()

    issue_idx(cstart, 0)
    wait_idx(0)
    issue_fetch(cstart, 0)
    issue_idx(cstart + 1, 1)

    def phase(c, b):
        nb = 1 - b
        wait_idx(nb)
        issue_fetch(jnp.minimum(c + 1, cmax), nb)
        wait_fetch(b)

        hi_mask = jnp.full((16,), 0xFFFF0000, jnp.uint32)

        def mul(r2, c2):
            for f in range(H // 16):
                wu = wv[b][r2, pl.ds(f * 16, 16)]
                we = plsc.bitcast(wu << 16, jnp.float32)
                wo = plsc.bitcast(wu & hi_mask, jnp.float32)
                xg[b][r2, pl.ds(f * 16, 16)] = (
                    xg[b][r2, pl.ds(f * 16, 16)] * we)
                xg[b][r2 + CH // 2, pl.ds(f * 16, 16)] = (
                    xg[b][r2 + CH // 2, pl.ds(f * 16, 16)] * wo)
            return c2

        lax.fori_loop(0, CH // 2, mul, 0)
        pltpu.sync_copy(xg[b], acc.at[col_v.at[b]], add=True)
        issue_idx(jnp.minimum(c + 2, cmax), b)

    def pair(t, carry):
        c = cstart + t * 2
        phase(c, 0)
        phase(c + 1, 1)
        return carry

    lax.fori_loop(0, cnum // 2, pair, 0)
    wait_fetch(0)
    wait_idx(1)
    plsc.subcore_barrier()
    pltpu.sync_copy(acc.at[pl.ds(sid * RPT, RPT)],
                    out_hbm.at[cid, pl.ds(sid * RPT, RPT)])


BE = 1280
NBE = E // BE


def _ewprep_body(ew2_ref, ew_ref, cc_ref):
    ew2 = ew2_ref[...]
    ew = jnp.sqrt(ew2)
    ew_ref[...] = ew
    cc_ref[...] = 0.5 * (jnp.cos(ew * (math.pi / CUTOFF)) + 1.0)


def _ewprep(ew2):
    shp = jax.ShapeDtypeStruct((E // 128, 128), jnp.float32)
    return pl.pallas_call(
        _ewprep_body,
        grid=(1,),
        in_specs=[pl.BlockSpec((E // 128, 128), lambda j: (0, 0))],
        out_specs=[pl.BlockSpec((E // 128, 128), lambda j: (0, 0))] * 2,
        out_shape=[shp, shp],
    )(ew2)


def _w_body(ew_ref, cc_ref, w1_ref, b1_ref, w2b_ref, b2_ref,
            o0_ref, o1_ref, o2_ref):
    ew = ew_ref[...]
    cc = cc_ref[...]
    step = CUTOFF / (G - 1)
    offs = lax.broadcasted_iota(jnp.int32, (1, G), 1).astype(jnp.float32) * step
    coeff = -0.5 / (step * step)
    ea = jnp.exp(coeff * (ew - offs) ** 2)
    eab = ea.astype(jnp.bfloat16)
    outs = (o0_ref, o1_ref, o2_ref)
    for i in range(NUM_INT):
        t = _ssp(jnp.dot(eab, w1_ref[i], preferred_element_type=jnp.float32)
                 + b1_ref[i])
        t = jnp.dot(t.astype(jnp.bfloat16), w2b_ref[i],
                    preferred_element_type=jnp.float32)
        t = (t + b2_ref[i]) * cc
        u = lax.bitcast_convert_type(t, jnp.uint32)
        tb = (u + 0x7FFF + ((u >> 16) & 1)) >> 16
        for cl in range(BE // CH):
            lo = tb[cl * CH:cl * CH + CH // 2, :]
            hi = tb[cl * CH + CH // 2:(cl + 1) * CH, :]
            outs[i][pl.ds(cl * (CH // 2), CH // 2), :] = lo | (hi << 16)


def _wkern(ew, cc, w1, b1, w2, b2):
    shp = jax.ShapeDtypeStruct((E // 2, H), jnp.uint32)
    return pl.pallas_call(
        _w_body,
        grid=(NBE,),
        in_specs=[
            pl.BlockSpec((BE, 1), lambda j: (j, 0)),
            pl.BlockSpec((BE, 1), lambda j: (j, 0)),
            pl.BlockSpec((NUM_INT, G, F), lambda j: (0, 0, 0)),
            pl.BlockSpec((NUM_INT, 1, F), lambda j: (0, 0, 0)),
            pl.BlockSpec((NUM_INT, F, F), lambda j: (0, 0, 0)),
            pl.BlockSpec((NUM_INT, 1, F), lambda j: (0, 0, 0)),
        ],
        out_specs=[pl.BlockSpec((BE // 2, H), lambda j: (j, 0))] * 3,
        out_shape=[shp, shp, shp],
    )(ew, cc, w1, b1, w2, b2)


BN = 1000
NBN = N // BN


def _prep_body(z_ref, emb_ref, lin1_ref, h_ref, xl_ref):
    z = z_ref[...]
    oh = (z == lax.broadcasted_iota(jnp.int32, (1, NZ), 1)).astype(jnp.float32)
    h = oh @ emb_ref[...]
    h_ref[...] = h
    xl_ref[...] = h @ lin1_ref[...]


def _prep(z, emb, lin1):
    return pl.pallas_call(
        _prep_body,
        grid=(NBN,),
        in_specs=[
            pl.BlockSpec((BN, 1), lambda j: (j, 0)),
            pl.BlockSpec((NZ, H), lambda j: (0, 0)),
            pl.BlockSpec((H, F), lambda j: (0, 0)),
        ],
        out_specs=[pl.BlockSpec((BN, H), lambda j: (j, 0))] * 2,
        out_shape=[jax.ShapeDtypeStruct((N, H), jnp.float32)] * 2,
    )(z, emb, lin1)


def _update_body(last, p_ref, h_ref, w2_ref, b2_ref, bw_ref, bb_ref,
                 lin1_ref, h_out_ref, xl_out_ref=None):
    agg = p_ref[0] + p_ref[1]
    x = agg @ w2_ref[...] + b2_ref[...]
    x = _ssp(x)
    x = x @ bw_ref[...] + bb_ref[...]
    hn = h_ref[...] + x
    h_out_ref[...] = hn
    if not last:
        xl_out_ref[...] = hn @ lin1_ref[...]


def _update(p, h, w2, b2, bw, bb, lin1, last):
    out_shape = [jax.ShapeDtypeStruct((N, H), jnp.float32)]
    out_specs = [pl.BlockSpec((BN, H), lambda j: (j, 0))]
    if not last:
        out_shape.append(jax.ShapeDtypeStruct((N, H), jnp.float32))
        out_specs.append(pl.BlockSpec((BN, H), lambda j: (j, 0)))
    return pl.pallas_call(
        functools.partial(_update_body, last),
        grid=(NBN,),
        in_specs=[
            pl.BlockSpec((NC, BN, H), lambda j: (0, j, 0)),
            pl.BlockSpec((BN, H), lambda j: (j, 0)),
            pl.BlockSpec((F, H), lambda j: (0, 0)),
            pl.BlockSpec((1, H), lambda j: (0, 0)),
            pl.BlockSpec((H, H), lambda j: (0, 0)),
            pl.BlockSpec((1, H), lambda j: (0, 0)),
            pl.BlockSpec((H, F), lambda j: (0, 0)),
        ],
        out_specs=out_specs,
        out_shape=out_shape,
    )(p, h, w2, b2, bw, bb, lin1)


def _head_body(h_ref, b_ref, o1w_ref, o1b_ref, o2w_ref, o2b_ref,
               out_ref, s_acc, c_acc):
    j = pl.program_id(0)

    @pl.when(j == 0)
    def _():
        s_acc[...] = jnp.zeros_like(s_acc)
        c_acc[...] = jnp.zeros_like(c_acc)

    t = _ssp(h_ref[...] @ o1w_ref[...] + o1b_ref[...])
    bt = b_ref[0]
    oh = (lax.broadcasted_iota(jnp.int32, (NUM_GRAPHS, 1), 0) == bt
          ).astype(jnp.float32)
    s_acc[...] += oh @ t
    c_acc[...] += jnp.sum(oh, axis=1, keepdims=True)

    @pl.when(j == pl.num_programs(0) - 1)
    def _():
        m = s_acc[...] / jnp.maximum(c_acc[...], 1.0)
        out_ref[...] = m @ o2w_ref[...] + o2b_ref[...]


def _head(h, batch3, o1w, o1b, o2w, o2b):
    return pl.pallas_call(
        _head_body,
        grid=(NBN,),
        in_specs=[
            pl.BlockSpec((BN, H), lambda j: (j, 0)),
            pl.BlockSpec((1, 1, BN), lambda j: (j, 0, 0)),
            pl.BlockSpec((H, H // 2), lambda j: (0, 0)),
            pl.BlockSpec((1, H // 2), lambda j: (0, 0)),
            pl.BlockSpec((H // 2, 1), lambda j: (0, 0)),
            pl.BlockSpec((1, 1), lambda j: (0, 0)),
        ],
        out_specs=pl.BlockSpec((NUM_GRAPHS, 1), lambda j: (0, 0)),
        out_shape=jax.ShapeDtypeStruct((NUM_GRAPHS, 1), jnp.float32),
        scratch_shapes=[
            pltpu.VMEM((NUM_GRAPHS, H // 2), jnp.float32),
            pltpu.VMEM((NUM_GRAPHS, 1), jnp.float32),
        ],
    )(h, batch3, o1w, o1b, o2w, o2b)


def kernel(z, pos, edge_index, edge_shift, batch, emb, mlp_w1, mlp_b1,
           mlp_w2, mlp_b2, conv_lin1_w, conv_lin2_w, conv_lin2_b,
           blk_lin_w, blk_lin_b, out1_w, out1_b, out2_w, out2_b):
    row = edge_index[0]
    col = edge_index[1]
    ew2 = _geom(pos.reshape(-1), row, col, edge_shift.reshape(-1))
    ew, cc = _ewprep(ew2.reshape(E // 128, 128))
    w_all = _wkern(ew.reshape(E, 1), cc.reshape(E, 1),
                   mlp_w1.astype(jnp.bfloat16), mlp_b1.reshape(NUM_INT, 1, F),
                   mlp_w2.astype(jnp.bfloat16), mlp_b2.reshape(NUM_INT, 1, F))
    h, xl = _prep(z.reshape(N, 1).astype(jnp.int32), emb, conv_lin1_w[0])
    for i in range(NUM_INT):
        p = _conv(xl, w_all[i], row, col)
        last = i == NUM_INT - 1
        res = _update(p, h, conv_lin2_w[i], conv_lin2_b[i].reshape(1, H),
                      blk_lin_w[i], blk_lin_b[i].reshape(1, H),
                      conv_lin1_w[(i + 1) % NUM_INT], last)
        if last:
            h = res[0]
        else:
            h, xl = res
    return _head(h, batch.reshape(NBN, 1, BN).astype(jnp.int32), out1_w,
                 out1_b.reshape(1, H // 2), out2_w, out2_b.reshape(1, 1))


# speedup vs baseline: 2.0296x; 1.0264x over previous
import functools
import math

import jax
import jax.numpy as jnp
from jax import lax
from jax.experimental import pallas as pl
from jax.experimental.pallas import tpu as pltpu
from jax.experimental.pallas import tpu_sc as plsc

N = 10000
E = 320000
H = 128
F = 128
G = 50
CUTOFF = 8.0
NUM_INT = 3
NUM_GRAPHS = 64
NZ = 100

NC = 2
NS = 16
NW = NC * NS
EPT = E // NW
CH = 128
NCH = E // CH
CPT = NCH // NW
RPT = 632
NPAD = RPT * NS

HP = H // 2

_mesh = plsc.VectorSubcoreMesh(core_axis_name="c", subcore_axis_name="s")
_sc_params = pltpu.CompilerParams(needs_layout_passes=False)


def _ssp(x):
    return jnp.maximum(x, 0.0) + jnp.log(1.0 + jnp.exp(-jnp.abs(x))) - math.log(2.0)


@functools.partial(
    pl.kernel,
    out_type=jax.ShapeDtypeStruct((E,), jnp.float32),
    mesh=_mesh,
    scratch_types=[
        pltpu.VMEM((3 * N,), jnp.float32),
        pltpu.VMEM((EPT,), jnp.int32),
        pltpu.VMEM((EPT,), jnp.int32),
        pltpu.VMEM((3 * EPT,), jnp.float32),
        pltpu.VMEM((EPT,), jnp.float32),
    ],
    compiler_params=_sc_params,
)
def _geom(pos_hbm, row_hbm, col_hbm, shift_hbm, ew2_hbm,
          pos_v, row_v, col_v, shift_v, ew2_v):
    wid = lax.axis_index("s") * NC + lax.axis_index("c")
    base = wid * EPT
    pltpu.sync_copy(pos_hbm, pos_v)
    pltpu.sync_copy(row_hbm.at[pl.ds(base, EPT)], row_v)
    pltpu.sync_copy(col_hbm.at[pl.ds(base, EPT)], col_v)
    pltpu.sync_copy(shift_hbm.at[pl.ds(3 * base, 3 * EPT)], shift_v)
    lanes = lax.iota(jnp.int32, 16)

    def body(k, carry):
        rv = row_v[pl.ds(k * 16, 16)] * 3
        cv = col_v[pl.ds(k * 16, 16)] * 3
        sbase = k * 48 + lanes * 3
        acc = None
        for j in range(3):
            pr = plsc.load_gather(pos_v, [rv + j])
            pc = plsc.load_gather(pos_v, [cv + j])
            sj = plsc.load_gather(shift_v, [sbase + j])
            d = pr - (pc + sj)
            acc = d * d if acc is None else acc + d * d
        ew2_v[pl.ds(k * 16, 16)] = acc
        return carry

    lax.fori_loop(0, EPT // 16, body, 0)
    pltpu.sync_copy(ew2_v, ew2_hbm.at[pl.ds(base, EPT)])


@functools.partial(
    pl.kernel,
    out_type=jax.ShapeDtypeStruct((NC, NPAD, H), jnp.float32),
    mesh=_mesh,
    scratch_types=[
        pltpu.VMEM((2, CH), jnp.int32),
        pltpu.VMEM((2, 2, CH // 2), jnp.int32),
        pltpu.VMEM((CH, H), jnp.float32),
        pltpu.VMEM((CH, H), jnp.float32),
        pltpu.VMEM((CH // 2, H), jnp.uint32),
        pltpu.VMEM((CH // 2, H), jnp.uint32),
        pltpu.VMEM_SHARED((NPAD, H), jnp.float32),
        pltpu.SemaphoreType.DMA,
        pltpu.SemaphoreType.DMA,
        pltpu.SemaphoreType.DMA,
        pltpu.SemaphoreType.DMA,
        pltpu.SemaphoreType.DMA,
        pltpu.SemaphoreType.DMA,
        pltpu.SemaphoreType.DMA,
        pltpu.SemaphoreType.DMA,
        pltpu.SemaphoreType.DMA,
    ],
    compiler_params=_sc_params,
)
def _conv(xl_hbm, w_hbm, row_hbm, col_hbm, out_hbm,
          row_v, col_v, xga, xgb, wva, wvb, acc,
          rs0, rs1, cs0, cs1, gs0, gs1, ws0, ws1, ssc):
    cid = lax.axis_index("c")
    sid = lax.axis_index("s")
    wid = sid * NC + cid
    xg = (xga, xgb)
    wv = (wva, wvb)
    rs = (rs0, rs1)
    cs = (cs0, cs1)
    gs = (gs0, gs1)
    ws = (ws0, ws1)

    def zbody(r, carry):
        for f in range(H // 16):
            xga[r, pl.ds(f * 16, 16)] = jnp.zeros((16,), jnp.float32)
        return carry

    lax.fori_loop(0, CH, zbody, 0)
    for t in range(RPT // CH):
        pltpu.sync_copy(xga, acc.at[pl.ds(sid * RPT + t * CH, CH)])
    pltpu.sync_copy(xga.at[pl.ds(0, RPT - (RPT // CH) * CH)],
                    acc.at[pl.ds(sid * RPT + (RPT // CH) * CH,
                                 RPT - (RPT // CH) * CH)])
    plsc.subcore_barrier()

    cstart = wid * CPT + 2 * jnp.minimum(wid, 2)
    cnum = CPT + jnp.where(wid < 2, 2, 0)
    cmax = cstart + cnum - 1

    def issue_idx(c, b):
        pltpu.async_copy(row_hbm.at[pl.ds(c * CH, CH)], row_v.at[b], rs[b])
        pltpu.async_copy(col_hbm.at[c], col_v.at[b], cs[b])

    def wait_idx(b):
        pltpu.make_async_copy(row_hbm.at[pl.ds(0, CH)], row_v.at[b], rs[b]).wait()
        pltpu.make_async_copy(col_hbm.at[0], col_v.at[b], cs[b]).wait()

    def issue_fetch(c, b):
        pltpu.async_copy(xl_hbm.at[row_v.at[b]], xg[b], gs[b])
        pltpu.async_copy(w_hbm.at[pl.ds(c * (CH // 2), CH // 2)], wv[b], ws[b])

    def wait_fetch(b):
        pltpu.make_async_copy(xl_hbm.at[row_v.at[b]], xg[b], gs[b]).wait()
        pltpu.make_async_copy(w_hbm.at[pl.ds(0, CH // 2)], wv[b], ws[b]).wait()

    issue_idx(cstart, 0)
    wait_idx(0)
    issue_fetch(cstart, 0)
    issue_idx(cstart + 1, 1)

    def phase(c, b):
        nb = 1 - b
        wait_idx(nb)
        issue_fetch(jnp.minimum(c + 1, cmax), nb)
        wait_fetch(b)

        hi_mask = jnp.full((16,), 0xFFFF0000, jnp.uint32)

        def mul_group(xoff, woff):
            def body(r2, c2):
                for f in range(H // 16):
                    wu = wv[b][woff + r2, pl.ds(f * 16, 16)]
                    we = plsc.bitcast(wu << 16, jnp.float32)
                    wo = plsc.bitcast(wu & hi_mask, jnp.float32)
                    xg[b][xoff + r2, pl.ds(f * 16, 16)] = (
                        xg[b][xoff + r2, pl.ds(f * 16, 16)] * we)
                    xg[b][xoff + 32 + r2, pl.ds(f * 16, 16)] = (
                        xg[b][xoff + 32 + r2, pl.ds(f * 16, 16)] * wo)
                return c2

            lax.fori_loop(0, 32, body, 0)

        hh = CH // 2
        mul_group(0, 0)
        pltpu.async_copy(xg[b].at[pl.ds(0, hh)], acc.at[col_v.at[b, 0]],
                         ssc, add=True)
        mul_group(hh, 32)
        pltpu.async_copy(xg[b].at[pl.ds(hh, hh)], acc.at[col_v.at[b, 1]],
                         ssc, add=True)
        for _ in range(2):
            pltpu.make_async_copy(xg[b].at[pl.ds(0, hh)],
                                  acc.at[col_v.at[b, 0]], ssc).wait()
        issue_idx(jnp.minimum(c + 2, cmax), b)

    def pair(t, carry):
        c = cstart + t * 2
        phase(c, 0)
        phase(c + 1, 1)
        return carry

    lax.fori_loop(0, cnum // 2, pair, 0)
    wait_fetch(0)
    wait_idx(1)
    plsc.subcore_barrier()
    pltpu.sync_copy(acc.at[pl.ds(sid * RPT, RPT)],
                    out_hbm.at[cid, pl.ds(sid * RPT, RPT)])


BE = 1280
NBE = E // BE


def _ewprep_body(ew2_ref, ew_ref, cc_ref):
    ew2 = ew2_ref[...]
    ew = jnp.sqrt(ew2)
    ew_ref[...] = ew
    cc_ref[...] = 0.5 * (jnp.cos(ew * (math.pi / CUTOFF)) + 1.0)


def _ewprep(ew2):
    shp = jax.ShapeDtypeStruct((E // 128, 128), jnp.float32)
    return pl.pallas_call(
        _ewprep_body,
        grid=(1,),
        in_specs=[pl.BlockSpec((E // 128, 128), lambda j: (0, 0))],
        out_specs=[pl.BlockSpec((E // 128, 128), lambda j: (0, 0))] * 2,
        out_shape=[shp, shp],
    )(ew2)


def _w_body(ew_ref, cc_ref, w1_ref, b1_ref, w2b_ref, b2_ref,
            o0_ref, o1_ref, o2_ref):
    ew = ew_ref[...]
    cc = cc_ref[...]
    step = CUTOFF / (G - 1)
    offs = lax.broadcasted_iota(jnp.int32, (1, G), 1).astype(jnp.float32) * step
    coeff = -0.5 / (step * step)
    ea = jnp.exp(coeff * (ew - offs) ** 2)
    eab = ea.astype(jnp.bfloat16)
    outs = (o0_ref, o1_ref, o2_ref)
    for i in range(NUM_INT):
        t = _ssp(jnp.dot(eab, w1_ref[i], preferred_element_type=jnp.float32)
                 + b1_ref[i])
        t = jnp.dot(t.astype(jnp.bfloat16), w2b_ref[i],
                    preferred_element_type=jnp.float32)
        t = (t + b2_ref[i]) * cc
        u = lax.bitcast_convert_type(t, jnp.uint32)
        tb = (u + 0x7FFF + ((u >> 16) & 1)) >> 16
        for g in range(BE // 64):
            lo = tb[g * 64:g * 64 + 32, :]
            hi = tb[g * 64 + 32:g * 64 + 64, :]
            outs[i][pl.ds(g * 32, 32), :] = lo | (hi << 16)


def _wkern(ew, cc, w1, b1, w2, b2):
    shp = jax.ShapeDtypeStruct((E // 2, H), jnp.uint32)
    return pl.pallas_call(
        _w_body,
        grid=(NBE,),
        in_specs=[
            pl.BlockSpec((BE, 1), lambda j: (j, 0)),
            pl.BlockSpec((BE, 1), lambda j: (j, 0)),
            pl.BlockSpec((NUM_INT, G, F), lambda j: (0, 0, 0)),
            pl.BlockSpec((NUM_INT, 1, F), lambda j: (0, 0, 0)),
            pl.BlockSpec((NUM_INT, F, F), lambda j: (0, 0, 0)),
            pl.BlockSpec((NUM_INT, 1, F), lambda j: (0, 0, 0)),
        ],
        out_specs=[pl.BlockSpec((BE // 2, H), lambda j: (j, 0))] * 3,
        out_shape=[shp, shp, shp],
    )(ew, cc, w1, b1, w2, b2)


BN = 1000
NBN = N // BN


def _prep_body(z_ref, emb_ref, lin1_ref, h_ref, xl_ref):
    z = z_ref[...]
    oh = (z == lax.broadcasted_iota(jnp.int32, (1, NZ), 1)).astype(jnp.float32)
    h = oh @ emb_ref[...]
    h_ref[...] = h
    xl_ref[...] = h @ lin1_ref[...]


def _prep(z, emb, lin1):
    return pl.pallas_call(
        _prep_body,
        grid=(NBN,),
        in_specs=[
            pl.BlockSpec((BN, 1), lambda j: (j, 0)),
            pl.BlockSpec((NZ, H), lambda j: (0, 0)),
            pl.BlockSpec((H, F), lambda j: (0, 0)),
        ],
        out_specs=[pl.BlockSpec((BN, H), lambda j: (j, 0))] * 2,
        out_shape=[jax.ShapeDtypeStruct((N, H), jnp.float32)] * 2,
    )(z, emb, lin1)


def _update_body(last, p_ref, h_ref, w2_ref, b2_ref, bw_ref, bb_ref,
                 lin1_ref, h_out_ref, xl_out_ref=None):
    agg = p_ref[0] + p_ref[1]
    x = agg @ w2_ref[...] + b2_ref[...]
    x = _ssp(x)
    x = x @ bw_ref[...] + bb_ref[...]
    hn = h_ref[...] + x
    h_out_ref[...] = hn
    if not last:
        xl_out_ref[...] = hn @ lin1_ref[...]


def _update(p, h, w2, b2, bw, bb, lin1, last):
    out_shape = [jax.ShapeDtypeStruct((N, H), jnp.float32)]
    out_specs = [pl.BlockSpec((BN, H), lambda j: (j, 0))]
    if not last:
        out_shape.append(jax.ShapeDtypeStruct((N, H), jnp.float32))
        out_specs.append(pl.BlockSpec((BN, H), lambda j: (j, 0)))
    return pl.pallas_call(
        functools.partial(_update_body, last),
        grid=(NBN,),
        in_specs=[
            pl.BlockSpec((NC, BN, H), lambda j: (0, j, 0)),
            pl.BlockSpec((BN, H), lambda j: (j, 0)),
            pl.BlockSpec((F, H), lambda j: (0, 0)),
            pl.BlockSpec((1, H), lambda j: (0, 0)),
            pl.BlockSpec((H, H), lambda j: (0, 0)),
            pl.BlockSpec((1, H), lambda j: (0, 0)),
            pl.BlockSpec((H, F), lambda j: (0, 0)),
        ],
        out_specs=out_specs,
        out_shape=out_shape,
    )(p, h, w2, b2, bw, bb, lin1)


def _head_body(h_ref, b_ref, o1w_ref, o1b_ref, o2w_ref, o2b_ref,
               out_ref, s_acc, c_acc):
    j = pl.program_id(0)

    @pl.when(j == 0)
    def _():
        s_acc[...] = jnp.zeros_like(s_acc)
        c_acc[...] = jnp.zeros_like(c_acc)

    t = _ssp(h_ref[...] @ o1w_ref[...] + o1b_ref[...])
    bt = b_ref[0]
    oh = (lax.broadcasted_iota(jnp.int32, (NUM_GRAPHS, 1), 0) == bt
          ).astype(jnp.float32)
    s_acc[...] += oh @ t
    c_acc[...] += jnp.sum(oh, axis=1, keepdims=True)

    @pl.when(j == pl.num_programs(0) - 1)
    def _():
        m = s_acc[...] / jnp.maximum(c_acc[...], 1.0)
        out_ref[...] = m @ o2w_ref[...] + o2b_ref[...]


def _head(h, batch3, o1w, o1b, o2w, o2b):
    return pl.pallas_call(
        _head_body,
        grid=(NBN,),
        in_specs=[
            pl.BlockSpec((BN, H), lambda j: (j, 0)),
            pl.BlockSpec((1, 1, BN), lambda j: (j, 0, 0)),
            pl.BlockSpec((H, H // 2), lambda j: (0, 0)),
            pl.BlockSpec((1, H // 2), lambda j: (0, 0)),
            pl.BlockSpec((H // 2, 1), lambda j: (0, 0)),
            pl.BlockSpec((1, 1), lambda j: (0, 0)),
        ],
        out_specs=pl.BlockSpec((NUM_GRAPHS, 1), lambda j: (0, 0)),
        out_shape=jax.ShapeDtypeStruct((NUM_GRAPHS, 1), jnp.float32),
        scratch_shapes=[
            pltpu.VMEM((NUM_GRAPHS, H // 2), jnp.float32),
            pltpu.VMEM((NUM_GRAPHS, 1), jnp.float32),
        ],
    )(h, batch3, o1w, o1b, o2w, o2b)


def kernel(z, pos, edge_index, edge_shift, batch, emb, mlp_w1, mlp_b1,
           mlp_w2, mlp_b2, conv_lin1_w, conv_lin2_w, conv_lin2_b,
           blk_lin_w, blk_lin_b, out1_w, out1_b, out2_w, out2_b):
    row = edge_index[0]
    col = edge_index[1]
    ew2 = _geom(pos.reshape(-1), row, col, edge_shift.reshape(-1))
    ew, cc = _ewprep(ew2.reshape(E // 128, 128))
    w_all = _wkern(ew.reshape(E, 1), cc.reshape(E, 1),
                   mlp_w1.astype(jnp.bfloat16), mlp_b1.reshape(NUM_INT, 1, F),
                   mlp_w2.astype(jnp.bfloat16), mlp_b2.reshape(NUM_INT, 1, F))
    h, xl = _prep(z.reshape(N, 1).astype(jnp.int32), emb, conv_lin1_w[0])
    col3 = col.reshape(NCH, 2, CH // 2)
    for i in range(NUM_INT):
        p = _conv(xl, w_all[i], row, col3)
        last = i == NUM_INT - 1
        res = _update(p, h, conv_lin2_w[i], conv_lin2_b[i].reshape(1, H),
                      blk_lin_w[i], blk_lin_b[i].reshape(1, H),
                      conv_lin1_w[(i + 1) % NUM_INT], last)
        if last:
            h = res[0]
        else:
            h, xl = res
    return _head(h, batch.reshape(NBN, 1, BN).astype(jnp.int32), out1_w,
                 out1_b.reshape(1, H // 2), out2_w, out2_b.reshape(1, 1))
